# baseline (speedup 1.0000x reference)
import functools

import jax
import jax.numpy as jnp
from jax import lax
from jax.experimental import pallas as pl
from jax.experimental.pallas import tpu as pltpu
from jax.experimental.pallas import tpu_sc as plsc

_R = 40
_S = 2


def _mm(a, b):
    return jnp.dot(a, b, preferred_element_type=jnp.float32)


def _hingcn_body(*refs):
    a0 = refs[:_S]
    a1 = refs[_S:2 * _S]
    (feats, wp0, wp1, wa0, wa1, ws0, ws1, av, beta_out, agg_out,
     oa0, oa1, cs0, cs1) = refs[2 * _S:]
    i = pl.program_id(0)

    @pl.when(i == 0)
    def _prep():
        oa0[...] = _mm(feats[...], _mm(wp1[...], wa0[...])).astype(jnp.bfloat16)
        oa1[...] = _mm(feats[...], _mm(wp1[...], wa1[...])).astype(jnp.bfloat16)
        cs0[...] = _mm(wp0[...], ws0[...])
        cs1[...] = _mm(wp0[...], ws1[...])

    for half in range(_S):
        r0 = (_S * i + half) * _R
        fblk = feats[pl.ds(r0, _R), :]
        hs = []
        for adj, oa, cs in ((a0[half], oa0, cs0), (a1[half], oa1, cs1)):
            pre = lax.dot_general(adj[...].astype(jnp.bfloat16), oa[...],
                                  (((1,), (0,)), ((), ())),
                                  preferred_element_type=jnp.float32)
            hs.append(jnp.maximum(pre + _mm(fblk, cs[...]), 0.0))
        h0, h1 = hs

        s0 = jnp.sum(jnp.tanh(h0) * av[...], axis=1, keepdims=True)
        s1 = jnp.sum(jnp.tanh(h1) * av[...], axis=1, keepdims=True)
        m = jnp.maximum(s0, s1)
        e0 = jnp.exp(s0 - m)
        e1 = jnp.exp(s1 - m)
        inv = 1.0 / (e0 + e1)
        b0 = e0 * inv
        b1 = e1 * inv

        beta_out[pl.ds(half * _R, _R), :] = jnp.concatenate([b0, b1], axis=1)
        agg_out[pl.ds(half * _R, _R), :] = b0 * h0 + b1 * h1


def _fc_body(g, wfc, bfc, out):
    out[...] = jnp.dot(g[...], wfc[...],
                       preferred_element_type=jnp.float32) + bfc[...]


def _sc_gather(table, idx):
    n, d = table.shape
    b = idx.shape[0]
    info = plsc.get_sparse_core_info()
    nw = info.num_cores * info.num_subcores
    bpw = b // nw
    mesh = plsc.VectorSubcoreMesh(core_axis_name="c", subcore_axis_name="s")

    @functools.partial(
        pl.kernel, mesh=mesh,
        out_type=jax.ShapeDtypeStruct((b, d), jnp.float32),
        scratch_types=[
            pltpu.VMEM((bpw,), jnp.int32),
            pltpu.VMEM((bpw, d), jnp.float32),
            pltpu.SemaphoreType.DMA,
        ],
    )
    def gk(table_hbm, idx_hbm, out_hbm, idx_v, rows_v, sem):
        wid = lax.axis_index("s") * info.num_cores + lax.axis_index("c")
        base = wid * bpw
        pltpu.sync_copy(idx_hbm.at[pl.ds(base, bpw)], idx_v)
        pltpu.async_copy(table_hbm.at[idx_v], rows_v, sem).wait()
        pltpu.sync_copy(rows_v, out_hbm.at[pl.ds(base, bpw)])

    return gk(table, idx)


def kernel(ids, feats, adjs_0, adjs_1, W_prep0, W_prep1, W_agg_0, W_agg_1,
           W_self_0, W_self_1, att_vec, W_fc, b_fc):
    n, d_feat = feats.shape
    prep = W_prep0.shape[1]
    hid = W_agg_0.shape[1]
    ncls = W_fc.shape[1]
    rows = _S * _R
    g = n // rows

    av2 = att_vec.reshape(1, hid)
    bfc2 = b_fc.reshape(1, ncls)

    full = lambda shape: pl.BlockSpec(shape, lambda i: (0, 0))
    adj_spec = [
        pl.BlockSpec((_R, n), functools.partial(
            lambda s, i: (_S * i + s, 0), s))
        for s in range(_S)
    ]
    beta_t, agg = pl.pallas_call(
        _hingcn_body,
        grid=(g,),
        in_specs=adj_spec + adj_spec + [
            full((n, d_feat)),
            full((d_feat, prep)), full((d_feat, prep)),
            full((prep, hid)), full((prep, hid)),
            full((prep, hid)), full((prep, hid)),
            full((1, hid)),
        ],
        out_specs=[
            pl.BlockSpec((rows, 2), lambda i: (i, 0)),
            pl.BlockSpec((rows, hid), lambda i: (i, 0)),
        ],
        out_shape=[
            jax.ShapeDtypeStruct((n, 2), jnp.float32),
            jax.ShapeDtypeStruct((n, hid), jnp.float32),
        ],
        scratch_shapes=[
            pltpu.VMEM((n, hid), jnp.bfloat16),
            pltpu.VMEM((n, hid), jnp.bfloat16),
            pltpu.VMEM((prep, hid), jnp.float32),
            pltpu.VMEM((prep, hid), jnp.float32),
        ],
        compiler_params=pltpu.CompilerParams(
            dimension_semantics=("arbitrary",)),
    )(*([adjs_0] * _S), *([adjs_1] * _S), feats, W_prep0, W_prep1,
      W_agg_0, W_agg_1, W_self_0, W_self_1, av2)

    gathered = _sc_gather(agg, ids.astype(jnp.int32))

    logits = pl.pallas_call(
        _fc_body,
        out_shape=jax.ShapeDtypeStruct((ids.shape[0], ncls), jnp.float32),
    )(gathered, W_fc, bfc2)

    return (logits, beta_t.T)


# speedup vs baseline: 1.2573x; 1.2573x over previous
import functools

import jax
import jax.numpy as jnp
from jax import lax
from jax.experimental import pallas as pl
from jax.experimental.pallas import tpu as pltpu
from jax.experimental.pallas import tpu_sc as plsc

_R = 200
_S = 1


def _mm(a, b):
    return jnp.dot(a, b, preferred_element_type=jnp.float32)


def _hingcn_body(*refs):
    a0 = refs[:_S]
    a1 = refs[_S:2 * _S]
    (feats, wp0, wp1, wa0, wa1, ws0, ws1, av, beta_out, agg_out,
     oa0, oa1, cs0, cs1) = refs[2 * _S:]
    i = pl.program_id(0)

    @pl.when(i == 0)
    def _prep():
        oa0[...] = _mm(feats[...], _mm(wp1[...], wa0[...])).astype(jnp.bfloat16)
        oa1[...] = _mm(feats[...], _mm(wp1[...], wa1[...])).astype(jnp.bfloat16)
        cs0[...] = _mm(wp0[...], ws0[...])
        cs1[...] = _mm(wp0[...], ws1[...])

    for half in range(_S):
        r0 = (_S * i + half) * _R
        fblk = feats[pl.ds(r0, _R), :]
        hs = []
        for adj, oa, cs in ((a0[half], oa0, cs0), (a1[half], oa1, cs1)):
            pre = lax.dot_general(adj[...].astype(jnp.bfloat16), oa[...],
                                  (((1,), (0,)), ((), ())),
                                  preferred_element_type=jnp.float32)
            hs.append(jnp.maximum(pre + _mm(fblk, cs[...]), 0.0))
        h0, h1 = hs

        s0 = jnp.sum(jnp.tanh(h0) * av[...], axis=1, keepdims=True)
        s1 = jnp.sum(jnp.tanh(h1) * av[...], axis=1, keepdims=True)
        m = jnp.maximum(s0, s1)
        e0 = jnp.exp(s0 - m)
        e1 = jnp.exp(s1 - m)
        inv = 1.0 / (e0 + e1)
        b0 = e0 * inv
        b1 = e1 * inv

        beta_out[pl.ds(half * _R, _R), :] = jnp.concatenate([b0, b1], axis=1)
        agg_out[pl.ds(half * _R, _R), :] = b0 * h0 + b1 * h1


def _fc_body(g, wfc, bfc, out):
    out[...] = jnp.dot(g[...], wfc[...],
                       preferred_element_type=jnp.float32) + bfc[...]


def _sc_gather(table, idx):
    n, d = table.shape
    b = idx.shape[0]
    info = plsc.get_sparse_core_info()
    nw = info.num_cores * info.num_subcores
    bpw = b // nw
    mesh = plsc.VectorSubcoreMesh(core_axis_name="c", subcore_axis_name="s")

    @functools.partial(
        pl.kernel, mesh=mesh,
        out_type=jax.ShapeDtypeStruct((b, d), jnp.float32),
        scratch_types=[
            pltpu.VMEM((bpw,), jnp.int32),
            pltpu.VMEM((bpw, d), jnp.float32),
            pltpu.SemaphoreType.DMA,
        ],
    )
    def gk(table_hbm, idx_hbm, out_hbm, idx_v, rows_v, sem):
        wid = lax.axis_index("s") * info.num_cores + lax.axis_index("c")
        base = wid * bpw
        pltpu.sync_copy(idx_hbm.at[pl.ds(base, bpw)], idx_v)
        pltpu.async_copy(table_hbm.at[idx_v], rows_v, sem).wait()
        pltpu.sync_copy(rows_v, out_hbm.at[pl.ds(base, bpw)])

    return gk(table, idx)


def kernel(ids, feats, adjs_0, adjs_1, W_prep0, W_prep1, W_agg_0, W_agg_1,
           W_self_0, W_self_1, att_vec, W_fc, b_fc):
    n, d_feat = feats.shape
    prep = W_prep0.shape[1]
    hid = W_agg_0.shape[1]
    ncls = W_fc.shape[1]
    rows = _S * _R
    g = n // rows

    av2 = att_vec.reshape(1, hid)
    bfc2 = b_fc.reshape(1, ncls)

    full = lambda shape: pl.BlockSpec(shape, lambda i: (0, 0))
    adj_spec = [
        pl.BlockSpec((_R, n), functools.partial(
            lambda s, i: (_S * i + s, 0), s))
        for s in range(_S)
    ]
    beta_t, agg = pl.pallas_call(
        _hingcn_body,
        grid=(g,),
        in_specs=adj_spec + adj_spec + [
            full((n, d_feat)),
            full((d_feat, prep)), full((d_feat, prep)),
            full((prep, hid)), full((prep, hid)),
            full((prep, hid)), full((prep, hid)),
            full((1, hid)),
        ],
        out_specs=[
            pl.BlockSpec((rows, 2), lambda i: (i, 0)),
            pl.BlockSpec((rows, hid), lambda i: (i, 0)),
        ],
        out_shape=[
            jax.ShapeDtypeStruct((n, 2), jnp.float32),
            jax.ShapeDtypeStruct((n, hid), jnp.float32),
        ],
        scratch_shapes=[
            pltpu.VMEM((n, hid), jnp.bfloat16),
            pltpu.VMEM((n, hid), jnp.bfloat16),
            pltpu.VMEM((prep, hid), jnp.float32),
            pltpu.VMEM((prep, hid), jnp.float32),
        ],
        compiler_params=pltpu.CompilerParams(
            dimension_semantics=("arbitrary",)),
    )(*([adjs_0] * _S), *([adjs_1] * _S), feats, W_prep0, W_prep1,
      W_agg_0, W_agg_1, W_self_0, W_self_1, av2)

    gathered = _sc_gather(agg, ids.astype(jnp.int32))

    logits = pl.pallas_call(
        _fc_body,
        out_shape=jax.ShapeDtypeStruct((ids.shape[0], ncls), jnp.float32),
    )(gathered, W_fc, bfc2)

    return (logits, beta_t.T)
